# speedup vs baseline: 12.7048x; 12.7048x over previous
import jax
import jax.numpy as jnp
from jax.experimental import pallas as pl
from jax.experimental.pallas import tpu as pltpu

_K = 256
_N = 32768
_R = 8


def _topk_mask_block(x_ref, o_ref):
    x = x_ref[...]
    i = jax.lax.bitcast_convert_type(x, jnp.uint32)
    sign = (i >> jnp.uint32(31)).astype(jnp.uint32)
    key = jnp.where(sign == jnp.uint32(1), ~i, i | jnp.uint32(0x80000000))

    t = jnp.zeros((_R, 1), jnp.uint32)
    for b in range(31, -1, -1):
        cand = t | jnp.uint32(1 << b)
        cnt = jnp.sum((key >= cand).astype(jnp.int32), axis=1, keepdims=True)
        t = jnp.where(cnt >= _K, cand, t)

    gt = key > t
    eq = key == t
    g = jnp.sum(gt.astype(jnp.int32), axis=1, keepdims=True)
    need = _K - g

    idx = jax.lax.broadcasted_iota(jnp.int32, (_R, _N), 1)
    c = jnp.zeros((_R, 1), jnp.int32)
    for b in range(14, -1, -1):
        cc = c | (1 << b)
        cnt = jnp.sum((eq & (idx < cc)).astype(jnp.int32), axis=1, keepdims=True)
        c = jnp.where(cnt < need, cc, c)

    mask = gt | (eq & (idx <= c))
    o_ref[...] = jnp.where(mask, x, 0.0)


def kernel(x):
    rows, n = x.shape
    grid = rows // _R
    return pl.pallas_call(
        _topk_mask_block,
        grid=(grid,),
        in_specs=[pl.BlockSpec((_R, n), lambda r: (r, 0))],
        out_specs=pl.BlockSpec((_R, n), lambda r: (r, 0)),
        out_shape=jax.ShapeDtypeStruct((rows, n), x.dtype),
    )(x)
